# baseline (speedup 1.0000x reference)
import functools

import jax
import jax.numpy as jnp
from jax import lax
from jax.experimental import pallas as pl
from jax.experimental.pallas import tpu as pltpu
from jax.experimental.pallas import tpu_sc as plsc

OLD = 100000
D = 128
NC = 2
NS = 16
NW = NC * NS
LANES = 16

TOTAL = 16384
RPT = TOTAL // NW
GCH = 128
NGC = RPT // GCH
EB = 64
DUMMY = TOTAL


def _mlp_body(nv_ref, nu_ref, ut_ref, w1_ref, b1_ref, w2_ref, b2_ref, o_ref):
  nv = nv_ref[...]
  nu = nu_ref[...]
  blk = nv.shape[0]
  iota = lax.broadcasted_iota(jnp.int32, (blk, 8), 1)
  onehot = (iota == nu).astype(jnp.float32)
  ut2 = ut_ref[:, 0:2]
  w1 = w1_ref[...]
  m = jnp.dot(ut2, w1[1:3, :], preferred_element_type=jnp.float32)
  h_pre = nv * w1[0:1, :] + jnp.dot(onehot, m, preferred_element_type=jnp.float32) + b1_ref[...]
  h = 0.5 * h_pre * (1.0 + lax.erf(h_pre * 0.7071067811865476))
  o_ref[...] = jnp.dot(h, w2_ref[...], preferred_element_type=jnp.float32) + b2_ref[...]


def _mlp_rows(num_values, num_units, unit_table, W1, b1, W2, b2, n_pad):
  blk = min(n_pad, 512)
  grid = n_pad // blk
  nv = jnp.zeros((n_pad, 1), jnp.float32).at[: num_values.shape[0], 0].set(num_values)
  nu = jnp.full((n_pad, 1), 0, jnp.int32).at[: num_units.shape[0], 0].set(num_units)
  ut_pad = jnp.zeros((8, 128), jnp.float32).at[:6, :2].set(unit_table)
  w1_pad = jnp.zeros((8, W1.shape[1]), jnp.float32).at[:3, :].set(W1)
  return pl.pallas_call(
      _mlp_body,
      grid=(grid,),
      in_specs=[
          pl.BlockSpec((blk, 1), lambda i: (i, 0)),
          pl.BlockSpec((blk, 1), lambda i: (i, 0)),
          pl.BlockSpec((8, 128), lambda i: (0, 0)),
          pl.BlockSpec((8, 512), lambda i: (0, 0)),
          pl.BlockSpec((1, 512), lambda i: (0, 0)),
          pl.BlockSpec((512, 128), lambda i: (0, 0)),
          pl.BlockSpec((1, 128), lambda i: (0, 0)),
      ],
      out_specs=pl.BlockSpec((blk, 128), lambda i: (i, 0)),
      out_shape=jax.ShapeDtypeStruct((n_pad, 128), jnp.float32),
  )(nv, nu, ut_pad, w1_pad, b1.reshape(1, 512), W2, b2.reshape(1, 128))


def _sc_kernel_factory(ls, ln):
  mesh = plsc.VectorSubcoreMesh(core_axis_name="c", subcore_axis_name="s")
  lt = ls + ln
  iters = max(1, (lt).bit_length())

  @functools.partial(
      pl.kernel,
      out_type=jax.ShapeDtypeStruct((TOTAL + NW, D), jnp.float32),
      mesh=mesh,
      compiler_params=pltpu.CompilerParams(needs_layout_passes=False),
      scratch_types=[
          pltpu.VMEM((RPT,), jnp.int32),
          pltpu.VMEM((RPT, D), jnp.float32),
          pltpu.VMEM((lt + EB,), jnp.int32),
          pltpu.VMEM((1, EB), jnp.int32),
          pltpu.VMEM((EB,), jnp.int32),
          pltpu.VMEM((EB, D), jnp.float32),
          pltpu.VMEM((EB + 8, D), jnp.float32),
          [pltpu.SemaphoreType.DMA] * 4,
          pltpu.SemaphoreType.DMA,
          pltpu.SemaphoreType.DMA,
      ],
  )
  def sc_kernel(ids_hbm, list_hbm, mlp_hbm, orig_hbm, new_hbm, out_hbm,
                ids_v, rows_v, list_v, tgt2d, sel_v, nrows_v, mrows_v,
                gsems, psem, ssem2):
    t = lax.axis_index("s") * NC + lax.axis_index("c")
    base = t * RPT

    with jax.named_scope("stage"):
      pltpu.sync_copy(ids_hbm.at[pl.ds(base, RPT)], ids_v)
      cps = []
      for j in range(NGC):
        cps.append(pltpu.async_copy(
            orig_hbm.at[ids_v.at[pl.ds(j * GCH, GCH)]],
            rows_v.at[pl.ds(j * GCH, GCH)], gsems[j]))
      pltpu.sync_copy(list_hbm, list_v)

    with jax.named_scope("search"):
      def lower_bound(lo0, hi0, key):
        def it(_, carry):
          lo, hi = carry
          mid = (lo + hi) // 2
          v = list_v[pl.ds(mid, LANES)][0]
          big = v >= key
          return jnp.where(big, lo, mid + 1), jnp.where(big, mid, hi)
        lo, _ = lax.fori_loop(
            0, iters, it, (jnp.int32(lo0), jnp.int32(hi0)))
        return lo

      lo_s = lower_bound(0, ls, base)
      hi_s = lower_bound(0, ls, base + RPT)
      lo_n = lower_bound(ls, lt, base)
      hi_n = lower_bound(ls, lt, base + RPT)

    with jax.named_scope("write_linear"):
      for j in range(NGC):
        cps[j].wait()
      pltpu.sync_copy(rows_v, out_hbm.at[pl.ds(base, RPT)])

    with jax.named_scope("patch"):
      def seg(lo, hi, is_num_seg):
        cnt = hi - lo
        nb2 = (cnt + EB - 1) // EB

        def bb(k, _):
          start = lo + jnp.maximum(0, jnp.minimum(k * EB, cnt - EB))
          for c in range(EB // LANES):
            e = start + c * LANES + lax.iota(jnp.int32, LANES)
            pos = list_v[pl.ds(start + c * LANES, LANES)]
            m = (e >= lo) & (e < hi)
            off = jnp.clip(pos - base, 0, RPT - 1)
            idv = plsc.load_gather(ids_v, [off], mask=m)
            sel_v[pl.ds(c * LANES, LANES)] = jnp.clip(idv - OLD, 0, 9999)
            tgt2d[0, pl.ds(c * LANES, LANES)] = jnp.where(m, pos, TOTAL + t)
          cpn = pltpu.async_copy(new_hbm.at[sel_v], nrows_v, psem)
          if is_num_seg:
            mstart = start - ls
            ma = pl.multiple_of((mstart // 8) * 8, 8)
            shift = mstart - ma
            pltpu.sync_copy(mlp_hbm.at[pl.ds(ma, EB + 8)], mrows_v)
          cpn.wait()
          if is_num_seg:
            def add_row(e2, _):
              for k2 in range(D // LANES):
                nrows_v[e2, pl.ds(k2 * LANES, LANES)] = (
                    nrows_v[e2, pl.ds(k2 * LANES, LANES)]
                    + mrows_v[e2 + shift, pl.ds(k2 * LANES, LANES)])
              return 0
            lax.fori_loop(0, EB, add_row, 0)
          pltpu.async_copy(nrows_v, out_hbm.at[tgt2d.at[0]], ssem2).wait()
          return 0
        lax.fori_loop(0, nb2, bb, 0)

      seg(lo_s, hi_s, False)
      seg(lo_n, hi_n, True)

  return sc_kernel


def _ceil(n, m):
  return max(m, (n + m - 1) // m * m)


def kernel(input_ids, stocks_pos, num_pos, num_values, num_units,
           orig_table, new_table, unit_table, W1, b1, W2, b2):
  ids_flat = input_ids.reshape(-1)
  n_s = stocks_pos.shape[0]
  n_n = num_pos.shape[0]
  ls, ln = _ceil(n_s, EB), _ceil(n_n, EB)
  lists = jnp.full((ls + ln + EB,), DUMMY, jnp.int32)
  lists = lists.at[:n_s].set(stocks_pos).at[ls:ls + n_n].set(num_pos)

  n_pad = _ceil(ln, 512) + 512
  mlp = _mlp_rows(num_values, num_units, unit_table, W1, b1, W2, b2, n_pad)

  sc = _sc_kernel_factory(ls, ln)
  out = sc(ids_flat, lists, mlp, orig_table, new_table)
  return out[:TOTAL].reshape(input_ids.shape[0], input_ids.shape[1], D)
